# baseline (speedup 1.0000x reference)
import jax
import jax.numpy as jnp
from jax import lax
from jax.experimental import pallas as pl
from jax.experimental.pallas import tpu as pltpu
from jax.experimental.pallas import tpu_sc as plsc

N = 10000
R = 16
D = 128
DH = 64
N_HALF = 2
E = 320000
NS = 16
C = 128
EP = 327680
PAD = EP - E
ECH = EP // C
ERE = E // C
EPW = EP // NS
MCH = EPW // C
K2 = 163840
L = 16
NROW = 632
BCH = 32
NBLK = MCH // BCH
ZF = 2048


def _splat(v):
    return jnp.full((L,), v, dtype=jnp.int32)


_GDN = lax.GatherDimensionNumbers(
    offset_dims=(), collapsed_slice_dims=(0,), start_index_map=(0,))


def _lane_bcast(v16, lane):
    return lax.gather(v16, _splat(lane)[:, None], _GDN, (1,),
                      mode=lax.GatherScatterMode.PROMISE_IN_BOUNDS)


def _make_sc_agg(first_layer):
    mesh = plsc.VectorSubcoreMesh(
        core_axis_name="c", subcore_axis_name="s", num_cores=1,
        num_subcores=NS)

    if first_layer:
        out_type = (jax.ShapeDtypeStruct((N_HALF, N, DH), jnp.float32),
                    jax.ShapeDtypeStruct((ECH, C), jnp.float32))
    else:
        out_type = jax.ShapeDtypeStruct((N_HALF, N, DH), jnp.float32)

    scratch = [
        pltpu.VMEM((BCH, C), jnp.int32),
        pltpu.VMEM((BCH, C), jnp.int32),
        pltpu.VMEM((BCH, C), jnp.int32),
        pltpu.VMEM((BCH, C), jnp.float32),
        pltpu.VMEM((C, DH), jnp.float32),
        pltpu.VMEM((C, DH), jnp.float32),
        pltpu.VMEM((8, DH), jnp.float32),
        pltpu.SemaphoreType.DMA,
        pltpu.SemaphoreType.DMA,
        pltpu.SemaphoreType.DMA,
        pltpu.SemaphoreType.DMA,
        pltpu.VMEM_SHARED((N + 8, DH), jnp.float32),
    ]
    if first_layer:
        scratch += [
            pltpu.VMEM((BCH, C), jnp.int32),
            pltpu.VMEM((C,), jnp.float32),
            pltpu.VMEM((ZF,), jnp.float32),
            pltpu.VMEM_SHARED((K2,), jnp.float32),
        ]

    def body(*refs):
        if first_layer:
            (src2, dst2, et2, Hall, part, w2_out,
             gbuf, dbuf, tbuf, wbuf, rows0, rows1, zrows,
             gsem0, gsem1, ssem0, ssem1, acc,
             kbuf, obuf, zflat, deg) = refs
            wsrc = w2_out
        else:
            (src2, dst2, et2, w2, Hall, part,
             gbuf, dbuf, tbuf, wbuf, rows0, rows1, zrows,
             gsem0, gsem1, ssem0, ssem1, acc) = refs
            wsrc = w2

        s = lax.axis_index("s")
        r0 = s * MCH

        for i in range(8):
            for j in range(DH // L):
                zrows[i, pl.ds(j * L, L)] = jnp.zeros((L,), jnp.float32)

        a0 = s * NROW
        acnt = lax.select(s == NS - 1, (N - (NS - 1) * NROW) // 8, NROW // 8)

        def _zacc(i, _):
            pltpu.sync_copy(zrows, acc.at[pl.ds(a0 + i * 8, 8)])
            return 0

        def _stage_keys(b):
            rb = r0 + b * BCH
            pltpu.sync_copy(dst2.at[pl.ds(rb, BCH)], dbuf)
            pltpu.sync_copy(et2.at[pl.ds(rb, BCH)], tbuf)

            def _keys(i, _1):
                for j in range(C // L):
                    sl = pl.ds(j * L, L)
                    kbuf[i, sl] = dbuf[i, sl] * R + tbuf[i, sl]
                return 0
            lax.fori_loop(0, BCH, _keys, 0)
            return rb

        if first_layer:
            def _zf(i, _):
                zflat[pl.ds(i * L, L)] = jnp.zeros((L,), jnp.float32)
                return 0
            lax.fori_loop(0, ZF // L, _zf, 0)
            for j in range(C // L):
                obuf[pl.ds(j * L, L)] = jnp.ones((L,), jnp.float32)

            def _zdeg(i, _):
                pltpu.sync_copy(
                    zflat, deg.at[pl.ds(s * (K2 // NS) + i * ZF, ZF)])
                return 0
            lax.fori_loop(0, (K2 // NS) // ZF, _zdeg, 0)
            plsc.subcore_barrier()

            def _dblk(b, _):
                _stage_keys(b)

                def _dsc(k, _1):
                    pltpu.async_copy(obuf, deg.at[kbuf.at[k]], gsem0,
                                     add=True)
                    return 0
                lax.fori_loop(0, BCH, _dsc, 0)

                def _ddr(k, _1):
                    pltpu.make_async_copy(
                        obuf, deg.at[kbuf.at[k]], gsem0).wait()
                    return 0
                lax.fori_loop(0, BCH, _ddr, 0)
                return 0
            lax.fori_loop(0, NBLK, _dblk, 0)
            plsc.subcore_barrier()

            def _wblk(b, _):
                rb = _stage_keys(b)

                def _wf(k, _1):
                    pltpu.async_copy(deg.at[kbuf.at[k]], wbuf.at[k], gsem0)
                    return 0
                lax.fori_loop(0, BCH, _wf, 0)

                def _wdr(k, _1):
                    pltpu.make_async_copy(
                        deg.at[kbuf.at[k]], wbuf.at[k], gsem0).wait()
                    return 0
                lax.fori_loop(0, BCH, _wdr, 0)

                def _wr(i, _1):
                    for j in range(C // L):
                        sl = pl.ds(j * L, L)
                        wbuf[i, sl] = 1.0 / wbuf[i, sl]
                    return 0
                lax.fori_loop(0, BCH, _wr, 0)
                pltpu.sync_copy(wbuf, w2_out.at[pl.ds(rb, BCH)])
                return 0
            lax.fori_loop(0, NBLK, _wblk, 0)

        for half in range(N_HALF):
            if half == 1:
                plsc.subcore_barrier()
            lax.fori_loop(0, acnt, _zacc, 0)
            plsc.subcore_barrier()

            def _blk(b, _):
                rb = r0 + b * BCH
                pltpu.sync_copy(dst2.at[pl.ds(rb, BCH)], dbuf)
                pltpu.sync_copy(et2.at[pl.ds(rb, BCH)], tbuf)
                pltpu.sync_copy(src2.at[pl.ds(rb, BCH)], gbuf)
                pltpu.sync_copy(wsrc.at[pl.ds(rb, BCH)], wbuf)

                def _gidx(i, _1):
                    for j in range(C // L):
                        sl = pl.ds(j * L, L)
                        gbuf[i, sl] = tbuf[i, sl] * N + gbuf[i, sl]
                    return 0
                lax.fori_loop(0, BCH, _gidx, 0)

                rr = (rows0, rows1)
                gs = (gsem0, gsem1)
                ss = (ssem0, ssem1)

                def _scale(rcur, k):
                    def _sg(g, _2):
                        wv16 = wbuf[k, pl.ds(pl.multiple_of(g * L, L), L)]
                        for i in range(L):
                            e = g * L + i
                            wv = _lane_bcast(wv16, i)
                            for j in range(DH // L):
                                sl = pl.ds(j * L, L)
                                rcur[e, sl] = rcur[e, sl] * wv
                        return 0
                    lax.fori_loop(0, C // L, _sg, 0)

                def _mainloop(k, _1):
                    pltpu.async_copy(
                        Hall.at[half].at[gbuf.at[k]], rows0, gsem0).wait()
                    _scale(rows0, k)
                    pltpu.sync_copy(rows0, acc.at[dbuf.at[k]], add=True)
                    return 0
                lax.fori_loop(0, BCH, _mainloop, 0)
                return 0
            lax.fori_loop(0, NBLK, _blk, 0)
            plsc.subcore_barrier()

            def _out(i, _):
                row = a0 + i * 8
                pltpu.sync_copy(acc.at[pl.ds(row, 8)],
                                part.at[half].at[pl.ds(row, 8)])
                return 0
            lax.fori_loop(0, acnt, _out, 0)

    return pl.kernel(
        body, out_type=out_type, mesh=mesh, scratch_types=scratch,
        compiler_params=pltpu.CompilerParams(use_tc_tiling_on_sc=False))


_BN = 1000


def _mm_body(x_ref, w_ref, b_ref, o_ref):
    res = jnp.dot(x_ref[...], w_ref[0],
                  preferred_element_type=jnp.float32) + b_ref[0]
    o_ref[0] = res[:, :DH]
    o_ref[1] = res[:, DH:]


def _tc_matmul(x, Wall, ball):
    nb = N // _BN
    return pl.pallas_call(
        _mm_body,
        grid=(nb, R + 1),
        in_specs=[
            pl.BlockSpec((_BN, D), lambda i, r: (i, 0)),
            pl.BlockSpec((1, D, D), lambda i, r: (r, 0, 0)),
            pl.BlockSpec((1, 1, D), lambda i, r: (r, 0, 0)),
        ],
        out_specs=pl.BlockSpec((N_HALF, _BN, DH),
                               lambda i, r: (0, r * nb + i, 0)),
        out_shape=jax.ShapeDtypeStruct((N_HALF, (R + 1) * N, DH),
                                       jnp.float32),
    )(x, Wall, ball)


def _make_add2(do_relu):
    def body(a_ref, p_ref, o_ref):
        lo = a_ref[0] + p_ref[0]
        hi = a_ref[1] + p_ref[1]
        t = jnp.concatenate([lo, hi], axis=1)
        o_ref[...] = jnp.maximum(t, 0.0) if do_relu else t

    def run(a, part):
        nb = N // _BN
        return pl.pallas_call(
            body,
            grid=(nb,),
            in_specs=[
                pl.BlockSpec((N_HALF, _BN, DH), lambda i: (0, i, 0)),
                pl.BlockSpec((N_HALF, _BN, DH), lambda i: (0, i, 0)),
            ],
            out_specs=pl.BlockSpec((_BN, D), lambda i: (i, 0)),
            out_shape=jax.ShapeDtypeStruct((N, D), jnp.float32),
        )(a, part)
    return run


_sc_agg1 = _make_sc_agg(True)
_sc_agg2 = _make_sc_agg(False)
_add2_relu = _make_add2(True)
_add2 = _make_add2(False)


def kernel(x, edge_index, edge_type, W1, root1, b1, W2, root2, b2, rel_emb):
    src = edge_index[0].astype(jnp.int32)
    dst = edge_index[1].astype(jnp.int32)
    et = edge_type.astype(jnp.int32)
    src2 = jnp.concatenate([src, jnp.zeros((PAD,), jnp.int32)]).reshape(
        ECH, C)
    dst2 = jnp.concatenate([dst, jnp.full((PAD,), N, jnp.int32)]).reshape(
        ECH, C)
    et2 = jnp.concatenate([et, jnp.zeros((PAD,), jnp.int32)]).reshape(ECH, C)

    Wall1 = jnp.concatenate([W1, root1[None]], axis=0)
    ball1 = jnp.zeros((R + 1, 1, D), jnp.float32).at[R, 0].set(b1)
    Wall2 = jnp.concatenate([W2, root2[None]], axis=0)
    ball2 = jnp.zeros((R + 1, 1, D), jnp.float32).at[R, 0].set(b2)

    Hall1 = _tc_matmul(x, Wall1, ball1)
    root1p = Hall1[:, R * N:]
    part1, w2 = _sc_agg1(src2, dst2, et2, Hall1)
    h = _add2_relu(root1p, part1)

    Hall2 = _tc_matmul(h, Wall2, ball2)
    root2p = Hall2[:, R * N:]
    part2 = _sc_agg2(src2, dst2, et2, w2, Hall2)
    out = _add2(root2p, part2)
    return (out, rel_emb)


# speedup vs baseline: 1.6344x; 1.6344x over previous
import jax
import jax.numpy as jnp
from jax import lax
from jax.experimental import pallas as pl
from jax.experimental.pallas import tpu as pltpu
from jax.experimental.pallas import tpu_sc as plsc

N = 10000
R = 16
D = 128
DH = 64
N_HALF = 2
E = 320000
NS = 16
C = 128
EP = 327680
PAD = EP - E
ECH = EP // C
ERE = E // C
EPW = EP // NS
MCH = EPW // C
K2 = 163840
L = 16
NROW = 632
BCH = 32
NBLK = MCH // BCH
ZF = 2048


def _splat(v):
    return jnp.full((L,), v, dtype=jnp.int32)


_GDN = lax.GatherDimensionNumbers(
    offset_dims=(), collapsed_slice_dims=(0,), start_index_map=(0,))


def _lane_bcast(v16, lane):
    return lax.gather(v16, _splat(lane)[:, None], _GDN, (1,),
                      mode=lax.GatherScatterMode.PROMISE_IN_BOUNDS)


def _make_sc_agg(first_layer):
    mesh = plsc.VectorSubcoreMesh(
        core_axis_name="c", subcore_axis_name="s", num_cores=1,
        num_subcores=NS)

    if first_layer:
        out_type = (jax.ShapeDtypeStruct((N_HALF, N, DH), jnp.float32),
                    jax.ShapeDtypeStruct((ECH, C), jnp.float32))
    else:
        out_type = jax.ShapeDtypeStruct((N_HALF, N, DH), jnp.float32)

    scratch = [
        pltpu.VMEM((BCH, C), jnp.int32),
        pltpu.VMEM((BCH, C), jnp.int32),
        pltpu.VMEM((BCH, C), jnp.int32),
        pltpu.VMEM((BCH, C), jnp.float32),
        pltpu.VMEM((C, DH), jnp.float32),
        pltpu.VMEM((C, DH), jnp.float32),
        pltpu.VMEM((8, DH), jnp.float32),
        pltpu.SemaphoreType.DMA,
        pltpu.SemaphoreType.DMA,
        pltpu.SemaphoreType.DMA,
        pltpu.SemaphoreType.DMA,
        pltpu.VMEM_SHARED((N + 8, DH), jnp.float32),
    ]
    if first_layer:
        scratch += [
            pltpu.VMEM((BCH, C), jnp.int32),
            pltpu.VMEM((C,), jnp.float32),
            pltpu.VMEM((ZF,), jnp.float32),
            pltpu.VMEM_SHARED((K2,), jnp.float32),
        ]

    def body(*refs):
        if first_layer:
            (src2, dst2, et2, Hall, part, w2_out,
             gbuf, dbuf, tbuf, wbuf, rows0, rows1, zrows,
             gsem0, gsem1, ssem0, ssem1, acc,
             kbuf, obuf, zflat, deg) = refs
            wsrc = w2_out
        else:
            (src2, dst2, et2, w2, Hall, part,
             gbuf, dbuf, tbuf, wbuf, rows0, rows1, zrows,
             gsem0, gsem1, ssem0, ssem1, acc) = refs
            wsrc = w2

        s = lax.axis_index("s")
        r0 = s * MCH

        for i in range(8):
            for j in range(DH // L):
                zrows[i, pl.ds(j * L, L)] = jnp.zeros((L,), jnp.float32)

        a0 = s * NROW
        acnt = lax.select(s == NS - 1, (N - (NS - 1) * NROW) // 8, NROW // 8)

        def _zacc(i, _):
            pltpu.sync_copy(zrows, acc.at[pl.ds(a0 + i * 8, 8)])
            return 0

        def _stage_keys(b):
            rb = r0 + b * BCH
            pltpu.sync_copy(dst2.at[pl.ds(rb, BCH)], dbuf)
            pltpu.sync_copy(et2.at[pl.ds(rb, BCH)], tbuf)

            def _keys(i, _1):
                for j in range(C // L):
                    sl = pl.ds(j * L, L)
                    kbuf[i, sl] = dbuf[i, sl] * R + tbuf[i, sl]
                return 0
            lax.fori_loop(0, BCH, _keys, 0)
            return rb

        if first_layer:
            def _zf(i, _):
                zflat[pl.ds(i * L, L)] = jnp.zeros((L,), jnp.float32)
                return 0
            lax.fori_loop(0, ZF // L, _zf, 0)
            for j in range(C // L):
                obuf[pl.ds(j * L, L)] = jnp.ones((L,), jnp.float32)

            def _zdeg(i, _):
                pltpu.sync_copy(
                    zflat, deg.at[pl.ds(s * (K2 // NS) + i * ZF, ZF)])
                return 0
            lax.fori_loop(0, (K2 // NS) // ZF, _zdeg, 0)
            plsc.subcore_barrier()

            def _dblk(b, _):
                _stage_keys(b)

                def _dsc(k, _1):
                    pltpu.async_copy(obuf, deg.at[kbuf.at[k]], gsem0,
                                     add=True)
                    return 0
                lax.fori_loop(0, BCH, _dsc, 0)

                def _ddr(k, _1):
                    pltpu.make_async_copy(
                        obuf, deg.at[kbuf.at[k]], gsem0).wait()
                    return 0
                lax.fori_loop(0, BCH, _ddr, 0)
                return 0
            lax.fori_loop(0, NBLK, _dblk, 0)
            plsc.subcore_barrier()

            def _wblk(b, _):
                rb = _stage_keys(b)

                def _wf(k, _1):
                    pltpu.async_copy(deg.at[kbuf.at[k]], wbuf.at[k], gsem0)
                    return 0
                lax.fori_loop(0, BCH, _wf, 0)

                def _wdr(k, _1):
                    pltpu.make_async_copy(
                        deg.at[kbuf.at[k]], wbuf.at[k], gsem0).wait()
                    return 0
                lax.fori_loop(0, BCH, _wdr, 0)

                def _wr(i, _1):
                    for j in range(C // L):
                        sl = pl.ds(j * L, L)
                        wbuf[i, sl] = 1.0 / wbuf[i, sl]
                    return 0
                lax.fori_loop(0, BCH, _wr, 0)
                pltpu.sync_copy(wbuf, w2_out.at[pl.ds(rb, BCH)])
                return 0
            lax.fori_loop(0, NBLK, _wblk, 0)

        for half in range(N_HALF):
            if half == 1:
                plsc.subcore_barrier()
            lax.fori_loop(0, acnt, _zacc, 0)
            plsc.subcore_barrier()

            def _blk(b, _):
                rb = r0 + b * BCH
                pltpu.sync_copy(dst2.at[pl.ds(rb, BCH)], dbuf)
                pltpu.sync_copy(et2.at[pl.ds(rb, BCH)], tbuf)
                pltpu.sync_copy(src2.at[pl.ds(rb, BCH)], gbuf)
                pltpu.sync_copy(wsrc.at[pl.ds(rb, BCH)], wbuf)

                def _gidx(i, _1):
                    for j in range(C // L):
                        sl = pl.ds(j * L, L)
                        gbuf[i, sl] = tbuf[i, sl] * N + gbuf[i, sl]
                    return 0
                lax.fori_loop(0, BCH, _gidx, 0)

                rr = (rows0, rows1)
                gs = (gsem0, gsem1)
                ss = (ssem0, ssem1)

                def _scale(rcur, k):
                    def _sg(g, _2):
                        wv16 = wbuf[k, pl.ds(pl.multiple_of(g * L, L), L)]

                        def _one(i, _3):
                            e = g * L + i
                            wv = _lane_bcast(wv16, i)
                            for j in range(DH // L):
                                sl = pl.ds(j * L, L)
                                rcur[e, sl] = rcur[e, sl] * wv
                            return 0
                        lax.fori_loop(0, L, _one, 0)
                        return 0
                    lax.fori_loop(0, C // L, _sg, 0)

                pltpu.async_copy(Hall.at[half].at[gbuf.at[0]], rows0, gsem0)

                def _grp(g2, _1):
                    for bix in range(2):
                        k = g2 * 2 + bix
                        cur, oth = bix, 1 - bix
                        pltpu.make_async_copy(
                            Hall.at[half].at[gbuf.at[k]], rr[cur],
                            gs[cur]).wait()
                        @pl.when(k > 0)
                        def _():
                            pltpu.make_async_copy(
                                rr[oth], acc.at[dbuf.at[k - 1]],
                                ss[oth]).wait()

                        @pl.when(k < BCH - 1)
                        def _():
                            pltpu.async_copy(
                                Hall.at[half].at[gbuf.at[k + 1]], rr[oth],
                                gs[oth])
                        _scale(rr[cur], k)
                        pltpu.async_copy(rr[cur], acc.at[dbuf.at[k]],
                                         ss[cur], add=True)
                    return 0
                lax.fori_loop(0, BCH // 2, _grp, 0)
                pltpu.make_async_copy(
                    rows1, acc.at[dbuf.at[BCH - 1]], ssem1).wait()
                return 0
            lax.fori_loop(0, NBLK, _blk, 0)
            plsc.subcore_barrier()

            def _out(i, _):
                row = a0 + i * 8
                pltpu.sync_copy(acc.at[pl.ds(row, 8)],
                                part.at[half].at[pl.ds(row, 8)])
                return 0
            lax.fori_loop(0, acnt, _out, 0)

    return pl.kernel(
        body, out_type=out_type, mesh=mesh, scratch_types=scratch,
        compiler_params=pltpu.CompilerParams(use_tc_tiling_on_sc=False))


_BN = 1000


def _mm_body(x_ref, w_ref, b_ref, o_ref):
    res = jnp.dot(x_ref[...], w_ref[0],
                  preferred_element_type=jnp.float32) + b_ref[0]
    o_ref[0] = res[:, :DH]
    o_ref[1] = res[:, DH:]


def _tc_matmul(x, Wall, ball):
    nb = N // _BN
    return pl.pallas_call(
        _mm_body,
        grid=(nb, R + 1),
        in_specs=[
            pl.BlockSpec((_BN, D), lambda i, r: (i, 0)),
            pl.BlockSpec((1, D, D), lambda i, r: (r, 0, 0)),
            pl.BlockSpec((1, 1, D), lambda i, r: (r, 0, 0)),
        ],
        out_specs=pl.BlockSpec((N_HALF, _BN, DH),
                               lambda i, r: (0, r * nb + i, 0)),
        out_shape=jax.ShapeDtypeStruct((N_HALF, (R + 1) * N, DH),
                                       jnp.float32),
    )(x, Wall, ball)


def _make_add2(do_relu):
    def body(a_ref, p_ref, o_ref):
        lo = a_ref[0] + p_ref[0]
        hi = a_ref[1] + p_ref[1]
        t = jnp.concatenate([lo, hi], axis=1)
        o_ref[...] = jnp.maximum(t, 0.0) if do_relu else t

    def run(a, part):
        nb = N // _BN
        return pl.pallas_call(
            body,
            grid=(nb,),
            in_specs=[
                pl.BlockSpec((N_HALF, _BN, DH), lambda i: (0, i, 0)),
                pl.BlockSpec((N_HALF, _BN, DH), lambda i: (0, i, 0)),
            ],
            out_specs=pl.BlockSpec((_BN, D), lambda i: (i, 0)),
            out_shape=jax.ShapeDtypeStruct((N, D), jnp.float32),
        )(a, part)
    return run


_sc_agg1 = _make_sc_agg(True)
_sc_agg2 = _make_sc_agg(False)
_add2_relu = _make_add2(True)
_add2 = _make_add2(False)


def kernel(x, edge_index, edge_type, W1, root1, b1, W2, root2, b2, rel_emb):
    src = edge_index[0].astype(jnp.int32)
    dst = edge_index[1].astype(jnp.int32)
    et = edge_type.astype(jnp.int32)
    src2 = jnp.concatenate([src, jnp.zeros((PAD,), jnp.int32)]).reshape(
        ECH, C)
    dst2 = jnp.concatenate([dst, jnp.full((PAD,), N, jnp.int32)]).reshape(
        ECH, C)
    et2 = jnp.concatenate([et, jnp.zeros((PAD,), jnp.int32)]).reshape(ECH, C)

    Wall1 = jnp.concatenate([W1, root1[None]], axis=0)
    ball1 = jnp.zeros((R + 1, 1, D), jnp.float32).at[R, 0].set(b1)
    Wall2 = jnp.concatenate([W2, root2[None]], axis=0)
    ball2 = jnp.zeros((R + 1, 1, D), jnp.float32).at[R, 0].set(b2)

    Hall1 = _tc_matmul(x, Wall1, ball1)
    root1p = Hall1[:, R * N:]
    part1, w2 = _sc_agg1(src2, dst2, et2, Hall1)
    h = _add2_relu(root1p, part1)

    Hall2 = _tc_matmul(h, Wall2, ball2)
    root2p = Hall2[:, R * N:]
    part2 = _sc_agg2(src2, dst2, et2, w2, Hall2)
    out = _add2(root2p, part2)
    return (out, rel_emb)


# speedup vs baseline: 1.6694x; 1.0214x over previous
import jax
import jax.numpy as jnp
from jax import lax
from jax.experimental import pallas as pl
from jax.experimental.pallas import tpu as pltpu
from jax.experimental.pallas import tpu_sc as plsc

N = 10000
R = 16
D = 128
DH = 64
N_HALF = 2
E = 320000
NS = 16
C = 128
EP = 327680
PAD = EP - E
ECH = EP // C
ERE = E // C
EPW = EP // NS
MCH = EPW // C
K2 = 163840
L = 16
NROW = 632
BCH = 32
NBLK = MCH // BCH
ZF = 2048


def _splat(v):
    return jnp.full((L,), v, dtype=jnp.int32)


_GDN = lax.GatherDimensionNumbers(
    offset_dims=(), collapsed_slice_dims=(0,), start_index_map=(0,))


def _lane_bcast(v16, lane):
    return lax.gather(v16, _splat(lane)[:, None], _GDN, (1,),
                      mode=lax.GatherScatterMode.PROMISE_IN_BOUNDS)


def _make_sc_agg(first_layer):
    mesh = plsc.VectorSubcoreMesh(
        core_axis_name="c", subcore_axis_name="s", num_cores=1,
        num_subcores=NS)

    if first_layer:
        out_type = (jax.ShapeDtypeStruct((N_HALF, N, DH), jnp.float32),
                    jax.ShapeDtypeStruct((ECH, C), jnp.float32))
    else:
        out_type = jax.ShapeDtypeStruct((N_HALF, N, DH), jnp.float32)

    scratch = [
        pltpu.VMEM((BCH, C), jnp.int32),
        pltpu.VMEM((BCH, C), jnp.int32),
        pltpu.VMEM((BCH, C), jnp.int32),
        pltpu.VMEM((BCH, C), jnp.float32),
        pltpu.VMEM((C, DH), jnp.float32),
        pltpu.VMEM((C, DH), jnp.float32),
        pltpu.VMEM((8, DH), jnp.float32),
        pltpu.SemaphoreType.DMA,
        pltpu.SemaphoreType.DMA,
        pltpu.SemaphoreType.DMA,
        pltpu.SemaphoreType.DMA,
        pltpu.VMEM_SHARED((N + 8, DH), jnp.float32),
    ]
    if first_layer:
        scratch += [
            pltpu.VMEM((BCH, C), jnp.int32),
            pltpu.VMEM((C,), jnp.float32),
            pltpu.VMEM((ZF,), jnp.float32),
            pltpu.VMEM_SHARED((K2,), jnp.float32),
        ]

    def body(*refs):
        if first_layer:
            (src2, dst2, et2, Hall, part, w2_out,
             gbuf, dbuf, tbuf, wbuf, rows0, rows1, zrows,
             gsem0, gsem1, ssem0, ssem1, acc,
             kbuf, obuf, zflat, deg) = refs
            wsrc = w2_out
        else:
            (src2, dst2, et2, w2, Hall, part,
             gbuf, dbuf, tbuf, wbuf, rows0, rows1, zrows,
             gsem0, gsem1, ssem0, ssem1, acc) = refs
            wsrc = w2

        s = lax.axis_index("s")
        r0 = s * MCH

        for i in range(8):
            for j in range(DH // L):
                zrows[i, pl.ds(j * L, L)] = jnp.zeros((L,), jnp.float32)

        a0 = s * NROW
        acnt = lax.select(s == NS - 1, (N - (NS - 1) * NROW) // 8, NROW // 8)

        def _zacc(i, _):
            pltpu.sync_copy(zrows, acc.at[pl.ds(a0 + i * 8, 8)])
            return 0

        def _stage_keys(b):
            rb = r0 + b * BCH
            pltpu.sync_copy(dst2.at[pl.ds(rb, BCH)], dbuf)
            pltpu.sync_copy(et2.at[pl.ds(rb, BCH)], tbuf)

            def _keys(i, _1):
                for j in range(C // L):
                    sl = pl.ds(j * L, L)
                    kbuf[i, sl] = dbuf[i, sl] * R + tbuf[i, sl]
                return 0
            lax.fori_loop(0, BCH, _keys, 0)
            return rb

        if first_layer:
            def _zf(i, _):
                zflat[pl.ds(i * L, L)] = jnp.zeros((L,), jnp.float32)
                return 0
            lax.fori_loop(0, ZF // L, _zf, 0)
            for j in range(C // L):
                obuf[pl.ds(j * L, L)] = jnp.ones((L,), jnp.float32)

            def _zdeg(i, _):
                pltpu.sync_copy(
                    zflat, deg.at[pl.ds(s * (K2 // NS) + i * ZF, ZF)])
                return 0
            lax.fori_loop(0, (K2 // NS) // ZF, _zdeg, 0)
            plsc.subcore_barrier()

            def _dblk(b, _):
                _stage_keys(b)

                def _dsc(k, _1):
                    pltpu.async_copy(obuf, deg.at[kbuf.at[k]], gsem0,
                                     add=True)
                    return 0
                lax.fori_loop(0, BCH, _dsc, 0)

                def _ddr(k, _1):
                    pltpu.make_async_copy(
                        obuf, deg.at[kbuf.at[k]], gsem0).wait()
                    return 0
                lax.fori_loop(0, BCH, _ddr, 0)
                return 0
            lax.fori_loop(0, NBLK, _dblk, 0)
            plsc.subcore_barrier()

            def _wblk(b, _):
                rb = _stage_keys(b)

                def _wf(k, _1):
                    pltpu.async_copy(deg.at[kbuf.at[k]], wbuf.at[k], gsem0)
                    return 0
                lax.fori_loop(0, BCH, _wf, 0)

                def _wdr(k, _1):
                    pltpu.make_async_copy(
                        deg.at[kbuf.at[k]], wbuf.at[k], gsem0).wait()
                    return 0
                lax.fori_loop(0, BCH, _wdr, 0)

                def _wr(i, _1):
                    for j in range(C // L):
                        sl = pl.ds(j * L, L)
                        wbuf[i, sl] = 1.0 / wbuf[i, sl]
                    return 0
                lax.fori_loop(0, BCH, _wr, 0)
                pltpu.sync_copy(wbuf, w2_out.at[pl.ds(rb, BCH)])
                return 0
            lax.fori_loop(0, NBLK, _wblk, 0)

        for half in range(N_HALF):
            if half == 1:
                plsc.subcore_barrier()
            lax.fori_loop(0, acnt, _zacc, 0)
            plsc.subcore_barrier()

            def _blk(b, _):
                rb = r0 + b * BCH
                pltpu.sync_copy(dst2.at[pl.ds(rb, BCH)], dbuf)
                pltpu.sync_copy(et2.at[pl.ds(rb, BCH)], tbuf)
                pltpu.sync_copy(src2.at[pl.ds(rb, BCH)], gbuf)
                pltpu.sync_copy(wsrc.at[pl.ds(rb, BCH)], wbuf)

                def _gidx(i, _1):
                    for j in range(C // L):
                        sl = pl.ds(j * L, L)
                        gbuf[i, sl] = tbuf[i, sl] * N + gbuf[i, sl]
                    return 0
                lax.fori_loop(0, BCH, _gidx, 0)

                rr = (rows0, rows1)
                gs = (gsem0, gsem1)
                ss = (ssem0, ssem1)

                def _scale(rcur, k):
                    def _sg(g, _2):
                        wv16 = wbuf[k, pl.ds(pl.multiple_of(g * L, L), L)]

                        def _one(q, _3):
                            for ii in range(4):
                                i = q * 4 + ii
                                e = g * L + i
                                wv = _lane_bcast(wv16, i)
                                for j in range(DH // L):
                                    sl = pl.ds(j * L, L)
                                    rcur[e, sl] = rcur[e, sl] * wv
                            return 0
                        lax.fori_loop(0, L // 4, _one, 0)
                        return 0
                    lax.fori_loop(0, C // L, _sg, 0)

                pltpu.async_copy(Hall.at[half].at[gbuf.at[0]], rows0, gsem0)

                def _grp(g2, _1):
                    for bix in range(2):
                        k = g2 * 2 + bix
                        cur, oth = bix, 1 - bix
                        pltpu.make_async_copy(
                            Hall.at[half].at[gbuf.at[k]], rr[cur],
                            gs[cur]).wait()
                        @pl.when(k > 0)
                        def _():
                            pltpu.make_async_copy(
                                rr[oth], acc.at[dbuf.at[k - 1]],
                                ss[oth]).wait()

                        @pl.when(k < BCH - 1)
                        def _():
                            pltpu.async_copy(
                                Hall.at[half].at[gbuf.at[k + 1]], rr[oth],
                                gs[oth])
                        _scale(rr[cur], k)
                        pltpu.async_copy(rr[cur], acc.at[dbuf.at[k]],
                                         ss[cur], add=True)
                    return 0
                lax.fori_loop(0, BCH // 2, _grp, 0)
                pltpu.make_async_copy(
                    rows1, acc.at[dbuf.at[BCH - 1]], ssem1).wait()
                return 0
            lax.fori_loop(0, NBLK, _blk, 0)
            plsc.subcore_barrier()

            def _out(i, _):
                row = a0 + i * 8
                pltpu.sync_copy(acc.at[pl.ds(row, 8)],
                                part.at[half].at[pl.ds(row, 8)])
                return 0
            lax.fori_loop(0, acnt, _out, 0)

    return pl.kernel(
        body, out_type=out_type, mesh=mesh, scratch_types=scratch,
        compiler_params=pltpu.CompilerParams(use_tc_tiling_on_sc=False))


_BN = 1000


def _mm_body(x_ref, w_ref, b_ref, o_ref):
    res = jnp.dot(x_ref[...].astype(jnp.bfloat16), w_ref[0],
                  preferred_element_type=jnp.float32) + b_ref[0]
    o_ref[0] = res[:, :DH]
    o_ref[1] = res[:, DH:]


def _tc_matmul(x, Wall, ball):
    nb = N // _BN
    return pl.pallas_call(
        _mm_body,
        grid=(nb, R + 1),
        in_specs=[
            pl.BlockSpec((_BN, D), lambda i, r: (i, 0)),
            pl.BlockSpec((1, D, D), lambda i, r: (r, 0, 0)),
            pl.BlockSpec((1, 1, D), lambda i, r: (r, 0, 0)),
        ],
        out_specs=pl.BlockSpec((N_HALF, _BN, DH),
                               lambda i, r: (0, r * nb + i, 0)),
        out_shape=jax.ShapeDtypeStruct((N_HALF, (R + 1) * N, DH),
                                       jnp.float32),
    )(x, Wall, ball)


def _make_add2(do_relu):
    def body(a_ref, p_ref, o_ref):
        lo = a_ref[0] + p_ref[0]
        hi = a_ref[1] + p_ref[1]
        t = jnp.concatenate([lo, hi], axis=1)
        o_ref[...] = jnp.maximum(t, 0.0) if do_relu else t

    def run(a, part):
        nb = N // _BN
        return pl.pallas_call(
            body,
            grid=(nb,),
            in_specs=[
                pl.BlockSpec((N_HALF, _BN, DH), lambda i: (0, i, 0)),
                pl.BlockSpec((N_HALF, _BN, DH), lambda i: (0, i, 0)),
            ],
            out_specs=pl.BlockSpec((_BN, D), lambda i: (i, 0)),
            out_shape=jax.ShapeDtypeStruct((N, D), jnp.float32),
        )(a, part)
    return run


_sc_agg1 = _make_sc_agg(True)
_sc_agg2 = _make_sc_agg(False)
_add2_relu = _make_add2(True)
_add2 = _make_add2(False)


def kernel(x, edge_index, edge_type, W1, root1, b1, W2, root2, b2, rel_emb):
    src = edge_index[0].astype(jnp.int32)
    dst = edge_index[1].astype(jnp.int32)
    et = edge_type.astype(jnp.int32)
    src2 = jnp.concatenate([src, jnp.zeros((PAD,), jnp.int32)]).reshape(
        ECH, C)
    dst2 = jnp.concatenate([dst, jnp.full((PAD,), N, jnp.int32)]).reshape(
        ECH, C)
    et2 = jnp.concatenate([et, jnp.zeros((PAD,), jnp.int32)]).reshape(ECH, C)

    Wall1 = jnp.concatenate([W1, root1[None]], axis=0)
    ball1 = jnp.zeros((R + 1, 1, D), jnp.float32).at[R, 0].set(b1)
    Wall2 = jnp.concatenate([W2, root2[None]], axis=0)
    ball2 = jnp.zeros((R + 1, 1, D), jnp.float32).at[R, 0].set(b2)

    Hall1 = _tc_matmul(x, Wall1.astype(jnp.bfloat16), ball1)
    root1p = Hall1[:, R * N:]
    part1, w2 = _sc_agg1(src2, dst2, et2, Hall1)
    h = _add2_relu(root1p, part1)

    Hall2 = _tc_matmul(h, Wall2.astype(jnp.bfloat16), ball2)
    root2p = Hall2[:, R * N:]
    part2 = _sc_agg2(src2, dst2, et2, w2, Hall2)
    out = _add2(root2p, part2)
    return (out, rel_emb)


# speedup vs baseline: 2.1043x; 1.2605x over previous
import jax
import jax.numpy as jnp
from jax import lax
from jax.experimental import pallas as pl
from jax.experimental.pallas import tpu as pltpu
from jax.experimental.pallas import tpu_sc as plsc

N = 10000
R = 16
D = 128
DH = 64
N_HALF = 2
E = 320000
NS = 16
C = 128
EP = 327680
PAD = EP - E
ECH = EP // C
ERE = E // C
EPW = EP // NS
MCH = EPW // C
K2 = 163840
L = 16
NROW = 632
BCH = 32
NBLK = MCH // BCH
ZF = 2048


def _splat(v):
    return jnp.full((L,), v, dtype=jnp.int32)


_GDN = lax.GatherDimensionNumbers(
    offset_dims=(), collapsed_slice_dims=(0,), start_index_map=(0,))


def _lane_bcast(v16, lane):
    return lax.gather(v16, _splat(lane)[:, None], _GDN, (1,),
                      mode=lax.GatherScatterMode.PROMISE_IN_BOUNDS)


def _make_sc_agg(first_layer):
    mesh = plsc.VectorSubcoreMesh(
        core_axis_name="c", subcore_axis_name="s", num_cores=1,
        num_subcores=NS)

    if first_layer:
        out_type = (jax.ShapeDtypeStruct((N, D), jnp.float32),
                    jax.ShapeDtypeStruct((ECH, C), jnp.float32))
    else:
        out_type = jax.ShapeDtypeStruct((N, D), jnp.float32)

    scratch = [
        pltpu.VMEM((BCH, C), jnp.int32),
        pltpu.VMEM((BCH, C), jnp.int32),
        pltpu.VMEM((BCH, C), jnp.int32),
        pltpu.VMEM((BCH, C), jnp.float32),
        pltpu.VMEM((C, DH), jnp.float32),
        pltpu.VMEM((C, DH), jnp.float32),
        pltpu.VMEM((8, DH), jnp.float32),
        pltpu.SemaphoreType.DMA,
        pltpu.SemaphoreType.DMA,
        pltpu.SemaphoreType.DMA,
        pltpu.SemaphoreType.DMA,
        pltpu.VMEM_SHARED((N + 8, DH), jnp.float32),
    ]
    if first_layer:
        scratch += [
            pltpu.VMEM((BCH, C), jnp.int32),
            pltpu.VMEM((C,), jnp.float32),
            pltpu.VMEM((ZF,), jnp.float32),
            pltpu.VMEM_SHARED((K2,), jnp.float32),
        ]

    def body(*refs):
        if first_layer:
            (src2, dst2, et2, Hall, part, w2_out,
             gbuf, dbuf, tbuf, wbuf, rows0, rows1, zrows,
             gsem0, gsem1, ssem0, ssem1, acc,
             kbuf, obuf, zflat, deg) = refs
            wsrc = w2_out
        else:
            (src2, dst2, et2, w2, Hall, part,
             gbuf, dbuf, tbuf, wbuf, rows0, rows1, zrows,
             gsem0, gsem1, ssem0, ssem1, acc) = refs
            wsrc = w2

        s = lax.axis_index("s")
        r0 = s * MCH

        for i in range(8):
            for j in range(DH // L):
                zrows[i, pl.ds(j * L, L)] = jnp.zeros((L,), jnp.float32)

        a0 = s * NROW
        acnt = lax.select(s == NS - 1, (N - (NS - 1) * NROW) // 8, NROW // 8)

        def _zacc(i, _):
            pltpu.sync_copy(zrows, acc.at[pl.ds(a0 + i * 8, 8)])
            return 0

        def _stage_keys(b):
            rb = r0 + b * BCH
            pltpu.sync_copy(dst2.at[pl.ds(rb, BCH)], dbuf)
            pltpu.sync_copy(et2.at[pl.ds(rb, BCH)], tbuf)

            def _keys(i, _1):
                for j in range(C // L):
                    sl = pl.ds(j * L, L)
                    kbuf[i, sl] = dbuf[i, sl] * R + tbuf[i, sl]
                return 0
            lax.fori_loop(0, BCH, _keys, 0)
            return rb

        if first_layer:
            def _zf(i, _):
                zflat[pl.ds(i * L, L)] = jnp.zeros((L,), jnp.float32)
                return 0
            lax.fori_loop(0, ZF // L, _zf, 0)
            for j in range(C // L):
                obuf[pl.ds(j * L, L)] = jnp.ones((L,), jnp.float32)

            def _zdeg(i, _):
                pltpu.sync_copy(
                    zflat, deg.at[pl.ds(s * (K2 // NS) + i * ZF, ZF)])
                return 0
            lax.fori_loop(0, (K2 // NS) // ZF, _zdeg, 0)
            plsc.subcore_barrier()

            def _dblk(b, _):
                _stage_keys(b)

                def _dsc(k, _1):
                    pltpu.async_copy(obuf, deg.at[kbuf.at[k]], gsem0,
                                     add=True)
                    return 0
                lax.fori_loop(0, BCH, _dsc, 0)

                def _ddr(k, _1):
                    pltpu.make_async_copy(
                        obuf, deg.at[kbuf.at[k]], gsem0).wait()
                    return 0
                lax.fori_loop(0, BCH, _ddr, 0)
                return 0
            lax.fori_loop(0, NBLK, _dblk, 0)
            plsc.subcore_barrier()

            def _wblk(b, _):
                rb = _stage_keys(b)

                def _wf(k, _1):
                    pltpu.async_copy(deg.at[kbuf.at[k]], wbuf.at[k], gsem0)
                    return 0
                lax.fori_loop(0, BCH, _wf, 0)

                def _wdr(k, _1):
                    pltpu.make_async_copy(
                        deg.at[kbuf.at[k]], wbuf.at[k], gsem0).wait()
                    return 0
                lax.fori_loop(0, BCH, _wdr, 0)

                def _wr(i, _1):
                    for j in range(C // L):
                        sl = pl.ds(j * L, L)
                        wbuf[i, sl] = 1.0 / wbuf[i, sl]
                    return 0
                lax.fori_loop(0, BCH, _wr, 0)
                pltpu.sync_copy(wbuf, w2_out.at[pl.ds(rb, BCH)])
                return 0
            lax.fori_loop(0, NBLK, _wblk, 0)

        for half in range(N_HALF):
            if half == 1:
                plsc.subcore_barrier()
            lax.fori_loop(0, acnt, _zacc, 0)
            plsc.subcore_barrier()

            def _blk(b, _):
                rb = r0 + b * BCH
                pltpu.sync_copy(dst2.at[pl.ds(rb, BCH)], dbuf)
                pltpu.sync_copy(et2.at[pl.ds(rb, BCH)], tbuf)
                pltpu.sync_copy(src2.at[pl.ds(rb, BCH)], gbuf)
                pltpu.sync_copy(wsrc.at[pl.ds(rb, BCH)], wbuf)

                def _gidx(i, _1):
                    for j in range(C // L):
                        sl = pl.ds(j * L, L)
                        gbuf[i, sl] = (tbuf[i, sl] * N + gbuf[i, sl]) * 2 \
                            + half
                    return 0
                lax.fori_loop(0, BCH, _gidx, 0)

                rr = (rows0, rows1)
                gs = (gsem0, gsem1)
                ss = (ssem0, ssem1)

                def _scale(rcur, k):
                    def _sg(g, _2):
                        wv16 = wbuf[k, pl.ds(pl.multiple_of(g * L, L), L)]

                        def _one(q, _3):
                            for ii in range(4):
                                i = q * 4 + ii
                                e = g * L + i
                                wv = _lane_bcast(wv16, i)
                                for j in range(DH // L):
                                    sl = pl.ds(j * L, L)
                                    rcur[e, sl] = rcur[e, sl] * wv
                            return 0
                        lax.fori_loop(0, L // 4, _one, 0)
                        return 0
                    lax.fori_loop(0, C // L, _sg, 0)

                pltpu.async_copy(Hall.at[gbuf.at[0]], rows0, gsem0)

                def _grp(g2, _1):
                    for bix in range(2):
                        k = g2 * 2 + bix
                        cur, oth = bix, 1 - bix
                        pltpu.make_async_copy(
                            Hall.at[gbuf.at[k]], rr[cur], gs[cur]).wait()
                        @pl.when(k > 0)
                        def _():
                            pltpu.make_async_copy(
                                rr[oth], acc.at[dbuf.at[k - 1]],
                                ss[oth]).wait()

                        @pl.when(k < BCH - 1)
                        def _():
                            pltpu.async_copy(
                                Hall.at[gbuf.at[k + 1]], rr[oth], gs[oth])
                        _scale(rr[cur], k)
                        pltpu.async_copy(rr[cur], acc.at[dbuf.at[k]],
                                         ss[cur], add=True)
                    return 0
                lax.fori_loop(0, BCH // 2, _grp, 0)
                pltpu.make_async_copy(
                    rows1, acc.at[dbuf.at[BCH - 1]], ssem1).wait()
                return 0
            lax.fori_loop(0, NBLK, _blk, 0)
            plsc.subcore_barrier()

            def _out(i, _):
                row = a0 + i * 8
                pltpu.sync_copy(
                    acc.at[pl.ds(row, 8)],
                    part.at[pl.ds(row, 8), pl.ds(half * DH, DH)])
                return 0
            lax.fori_loop(0, acnt, _out, 0)

    return pl.kernel(
        body, out_type=out_type, mesh=mesh, scratch_types=scratch,
        compiler_params=pltpu.CompilerParams(use_tc_tiling_on_sc=False))


_BN = 1000


def _mm_body(x_ref, w_ref, b_ref, o_ref):
    o_ref[...] = jnp.dot(x_ref[...].astype(jnp.bfloat16), w_ref[0],
                         preferred_element_type=jnp.float32) + b_ref[0]


def _tc_matmul(x, Wall, ball):
    nb = N // _BN
    return pl.pallas_call(
        _mm_body,
        grid=(nb, R + 1),
        in_specs=[
            pl.BlockSpec((_BN, D), lambda i, r: (i, 0)),
            pl.BlockSpec((1, D, D), lambda i, r: (r, 0, 0)),
            pl.BlockSpec((1, 1, D), lambda i, r: (r, 0, 0)),
        ],
        out_specs=pl.BlockSpec((_BN, D), lambda i, r: (r * nb + i, 0)),
        out_shape=jax.ShapeDtypeStruct(((R + 1) * N, D), jnp.float32),
    )(x, Wall, ball)


def _make_add2(do_relu):
    def body(a_ref, p_ref, o_ref):
        t = a_ref[...] + p_ref[...]
        o_ref[...] = jnp.maximum(t, 0.0) if do_relu else t

    def run(a, part):
        nb = N // _BN
        return pl.pallas_call(
            body,
            grid=(nb,),
            in_specs=[
                pl.BlockSpec((_BN, D), lambda i: (i, 0)),
                pl.BlockSpec((_BN, D), lambda i: (i, 0)),
            ],
            out_specs=pl.BlockSpec((_BN, D), lambda i: (i, 0)),
            out_shape=jax.ShapeDtypeStruct((N, D), jnp.float32),
        )(a, part)
    return run


_sc_agg1 = _make_sc_agg(True)
_sc_agg2 = _make_sc_agg(False)
_add2_relu = _make_add2(True)
_add2 = _make_add2(False)


def kernel(x, edge_index, edge_type, W1, root1, b1, W2, root2, b2, rel_emb):
    src = edge_index[0].astype(jnp.int32)
    dst = edge_index[1].astype(jnp.int32)
    et = edge_type.astype(jnp.int32)
    src2 = jnp.concatenate([src, jnp.zeros((PAD,), jnp.int32)]).reshape(
        ECH, C)
    dst2 = jnp.concatenate([dst, jnp.full((PAD,), N, jnp.int32)]).reshape(
        ECH, C)
    et2 = jnp.concatenate([et, jnp.zeros((PAD,), jnp.int32)]).reshape(ECH, C)

    Wall1 = jnp.concatenate([W1, root1[None]], axis=0)
    ball1 = jnp.zeros((R + 1, 1, D), jnp.float32).at[R, 0].set(b1)
    Wall2 = jnp.concatenate([W2, root2[None]], axis=0)
    ball2 = jnp.zeros((R + 1, 1, D), jnp.float32).at[R, 0].set(b2)

    Hall1 = _tc_matmul(x, Wall1.astype(jnp.bfloat16), ball1)
    root1p = Hall1[R * N:]
    part1, w2 = _sc_agg1(src2, dst2, et2,
                         Hall1.reshape((R + 1) * N * 2, DH))
    h = _add2_relu(root1p, part1)

    Hall2 = _tc_matmul(h, Wall2.astype(jnp.bfloat16), ball2)
    root2p = Hall2[R * N:]
    part2 = _sc_agg2(src2, dst2, et2, w2,
                     Hall2.reshape((R + 1) * N * 2, DH))
    out = _add2(root2p, part2)
    return (out, rel_emb)
